# baseline (speedup 1.0000x reference)
import functools

import jax
import jax.numpy as jnp
import numpy as np
from jax.experimental import pallas as pl
from jax.experimental.pallas import tpu as pltpu

S, D, H, E = 2048, 768, 12, 64
DH = D // H
FF = 2 * D
TOPK = 1
CAP = int(1.25 * S / E)
EPS = 1e-6
ZW = 1e-3
NEG = -1e9
HALF = DH // 2

SBLK = 256
NS = S // SBLK
NPAD = ((E * CAP + 8) // 8) * 8

HIGH = jax.lax.Precision.HIGHEST


def _dot(a, b, dims):
    return jax.lax.dot_general(a.astype(jnp.bfloat16), b.astype(jnp.bfloat16),
                               dims, preferred_element_type=jnp.float32)


def _rms(x, g):
    return x * jax.lax.rsqrt(jnp.mean(x * x, axis=-1, keepdims=True) + EPS) * g


def _rope_swap(x):
    parts = []
    for h in range(H):
        parts.append(x[:, h * DH + HALF:h * DH + DH])
        parts.append(x[:, h * DH:h * DH + HALF])
    return jnp.concatenate(parts, axis=1)


def _k123_block(x_ref, g1_ref, wqkv_ref, cos_ref, sin_ref, wo_ref, g2_ref,
                wg_ref, h_ref, xt_ref, lg_ref, ks_ref, vs_ref, sc_ref):
    i = pl.program_id(0)
    xn = _rms(x_ref[...], g1_ref[...])
    qkv = _dot(xn, wqkv_ref[...], (((1,), (0,)), ((), ())))
    q, k, v = qkv[:, :D], qkv[:, D:2 * D], qkv[:, 2 * D:]
    c, s = cos_ref[...], sin_ref[...]
    q = q * c + _rope_swap(q) * s
    ks_ref[pl.ds(i * SBLK, SBLK), :] = k * c + _rope_swap(k) * s
    vs_ref[pl.ds(i * SBLK, SBLK), :] = v

    row_g = jax.lax.broadcasted_iota(jnp.int32, (SBLK, SBLK), 0) + i * SBLK
    col_l = jax.lax.broadcasted_iota(jnp.int32, (SBLK, SBLK), 1)
    scale = 1.0 / np.sqrt(DH)
    o_parts = []
    for h in range(H):
        sl = slice(h * DH, (h + 1) * DH)
        qh = q[:, sl]

        def p1(j, m):
            sc = _dot(qh, ks_ref[pl.ds(j * SBLK, SBLK), sl],
                      (((1,), (1,)), ((), ()))) * scale
            sc = jnp.where(col_l + j * SBLK <= row_g, sc, NEG)
            sc_ref[:, pl.ds(j * SBLK, SBLK)] = sc
            return jnp.maximum(m, jnp.max(sc, axis=1, keepdims=True))

        m = jax.lax.fori_loop(0, i + 1, p1, jnp.full((SBLK, 1), NEG,
                                                     jnp.float32))

        def p2(j, carry):
            l, acc = carry
            p = jnp.exp(sc_ref[:, pl.ds(j * SBLK, SBLK)] - m)
            l = l + jnp.sum(p, axis=1, keepdims=True)
            acc = acc + _dot(p, vs_ref[pl.ds(j * SBLK, SBLK), sl],
                             (((1,), (0,)), ((), ())))
            return l, acc

        l, acc = jax.lax.fori_loop(
            0, i + 1, p2,
            (jnp.zeros((SBLK, 1), jnp.float32),
             jnp.zeros((SBLK, DH), jnp.float32)))
        o_parts.append(acc / l)
    o = jnp.concatenate(o_parts, axis=1)
    hh = x_ref[...] + _dot(o, wo_ref[...], (((1,), (0,)), ((), ())))
    xt = _rms(hh, g2_ref[...])
    h_ref[...] = hh
    xt_ref[...] = xt
    lg_ref[...] = _dot(xt, wg_ref[...], (((1,), (0,)), ((), ())))


def _k4_route(lg_ref, flat_ref, p_ref, aux_ref):
    CH = 128
    nch = S // CH
    r_i = jax.lax.broadcasted_iota(jnp.int32, (CH, CH), 0)
    c_i = jax.lax.broadcasted_iota(jnp.int32, (CH, CH), 1)
    tril = (r_i >= c_i).astype(jnp.float32)
    iota_e = jax.lax.broadcasted_iota(jnp.int32, (CH, E), 1)
    base = jnp.zeros((1, E), jnp.float32)
    p_acc = jnp.zeros((1, E), jnp.float32)
    lse2 = jnp.zeros((1, 1), jnp.float32)
    for c in range(nch):
        lg = lg_ref[c * CH:(c + 1) * CH, :]
        m = jnp.max(lg, axis=1, keepdims=True)
        ex = jnp.exp(lg - m)
        se = jnp.sum(ex, axis=1, keepdims=True)
        p_acc = p_acc + jnp.sum(ex / se, axis=0, keepdims=True)
        lse = m + jnp.log(se)
        lse2 = lse2 + jnp.sum(lse * lse, axis=0, keepdims=True)
        eidx = jnp.min(jnp.where(lg == m, iota_e, E), axis=1, keepdims=True)
        oh = (iota_e == eidx).astype(jnp.float32)
        csum = jax.lax.dot_general(tril, oh, (((1,), (0,)), ((), ())))
        pos = jnp.sum((csum + base) * oh, axis=1, keepdims=True).astype(
            jnp.int32) - 1
        keep = pos < CAP
        flat_ref[c * CH:(c + 1) * CH, :] = jnp.where(
            keep, eidx * CAP + pos, E * CAP)
        base = base + jnp.sum(oh, axis=0, keepdims=True)
    f = base * (1.0 / S)
    p = p_acc * (1.0 / S)
    p_ref[...] = p
    aux_ref[...] = (E * jnp.sum(f * p, keepdims=True).reshape(1, 1)
                    + ZW * lse2 * (1.0 / S))


def _k5_scatter(flat_ref, xt_ref, ein_ref):
    g = pl.program_id(0)

    @pl.when(g == 0)
    def _zero():
        ein_ref[...] = jnp.zeros((NPAD, D), jnp.float32)

    def body(i, _):
        idx = flat_ref[g * SBLK + i]
        ein_ref[pl.ds(idx, 1), :] = xt_ref[pl.ds(i, 1), :]
        return 0

    jax.lax.fori_loop(0, SBLK, body, 0)


def _k6_ffn(ein_ref, w1_ref, w2_ref, eout_ref):
    x = ein_ref[...]
    h1 = _dot(x, w1_ref[0], (((1,), (0,)), ((), ())))
    h1 = jax.nn.gelu(h1)
    eout_ref[...] = _dot(h1, w2_ref[0], (((1,), (0,)), ((), ())))


def _k7_gather(flat_ref, h_ref, eout_ref, out_ref):
    g = pl.program_id(0)
    out_ref[...] = h_ref[...]

    def body(i, _):
        idx = flat_ref[g * SBLK + i]
        safe = jnp.minimum(idx, E * CAP - 1)
        w = (idx < E * CAP).astype(jnp.float32)
        out_ref[pl.ds(i, 1), :] += eout_ref[pl.ds(safe, 1), :] * w
        return 0

    jax.lax.fori_loop(0, SBLK, body, 0)


@jax.jit
def kernel(x, ln1_g, ln2_g, Wqkv, Wo, Wg, W1, W2):
    xt2 = x.reshape(S, D)
    g1 = ln1_g.reshape(1, D)
    g2 = ln2_g.reshape(1, D)

    freqs = 1.0 / (10000.0 ** (jnp.arange(HALF, dtype=jnp.float32) / HALF))
    ang = jnp.arange(S, dtype=jnp.float32)[:, None] * freqs[None, :]
    cos_t = jnp.tile(jnp.concatenate([jnp.cos(ang)] * 2, axis=1), (1, H))
    sin_t = jnp.tile(
        jnp.concatenate([-jnp.sin(ang), jnp.sin(ang)], axis=1), (1, H))

    blk = lambda idx: pl.BlockSpec((SBLK, D), idx)
    h, xt, lg = pl.pallas_call(
        _k123_block,
        grid=(NS,),
        in_specs=[
            blk(lambda i: (i, 0)),
            pl.BlockSpec((1, D), lambda i: (0, 0)),
            pl.BlockSpec((D, 3 * D), lambda i: (0, 0)),
            blk(lambda i: (i, 0)),
            blk(lambda i: (i, 0)),
            pl.BlockSpec((D, D), lambda i: (0, 0)),
            pl.BlockSpec((1, D), lambda i: (0, 0)),
            pl.BlockSpec((D, E), lambda i: (0, 0)),
        ],
        out_specs=[blk(lambda i: (i, 0)), blk(lambda i: (i, 0)),
                   pl.BlockSpec((SBLK, E), lambda i: (i, 0))],
        out_shape=[jax.ShapeDtypeStruct((S, D), jnp.float32),
                   jax.ShapeDtypeStruct((S, D), jnp.float32),
                   jax.ShapeDtypeStruct((S, E), jnp.float32)],
        scratch_shapes=[pltpu.VMEM((S, D), jnp.float32),
                        pltpu.VMEM((S, D), jnp.float32),
                        pltpu.VMEM((SBLK, S), jnp.float32)],
    )(xt2, g1, Wqkv, cos_t, sin_t, Wo, g2, Wg)

    flat, p, aux = pl.pallas_call(
        _k4_route,
        in_specs=[pl.BlockSpec((S, E), lambda: (0, 0))],
        out_specs=[pl.BlockSpec((S, 1), lambda: (0, 0)),
                   pl.BlockSpec((1, E), lambda: (0, 0)),
                   pl.BlockSpec((1, 1), lambda: (0, 0))],
        out_shape=[jax.ShapeDtypeStruct((S, 1), jnp.int32),
                   jax.ShapeDtypeStruct((1, E), jnp.float32),
                   jax.ShapeDtypeStruct((1, 1), jnp.float32)],
    )(lg)

    flat1 = flat.reshape(S)

    ein = pl.pallas_call(
        _k5_scatter,
        grid_spec=pltpu.PrefetchScalarGridSpec(
            num_scalar_prefetch=1,
            grid=(NS,),
            in_specs=[pl.BlockSpec((SBLK, D), lambda i, s: (i, 0))],
            out_specs=pl.BlockSpec((NPAD, D), lambda i, s: (0, 0)),
        ),
        out_shape=jax.ShapeDtypeStruct((NPAD, D), jnp.float32),
    )(flat1, xt)

    eout = pl.pallas_call(
        _k6_ffn,
        grid=(E,),
        in_specs=[
            pl.BlockSpec((CAP, D), lambda e: (e, 0)),
            pl.BlockSpec((1, D, FF), lambda e: (e, 0, 0)),
            pl.BlockSpec((1, FF, D), lambda e: (e, 0, 0)),
        ],
        out_specs=pl.BlockSpec((CAP, D), lambda e: (e, 0)),
        out_shape=jax.ShapeDtypeStruct((E * CAP, D), jnp.float32),
    )(ein, W1, W2)

    out = pl.pallas_call(
        _k7_gather,
        grid_spec=pltpu.PrefetchScalarGridSpec(
            num_scalar_prefetch=1,
            grid=(NS,),
            in_specs=[pl.BlockSpec((SBLK, D), lambda i, s: (i, 0)),
                      pl.BlockSpec((E * CAP, D), lambda i, s: (0, 0))],
            out_specs=pl.BlockSpec((SBLK, D), lambda i, s: (i, 0)),
        ),
        out_shape=jax.ShapeDtypeStruct((S, D), jnp.float32),
    )(flat1, h, eout)

    return out.reshape(1, S, D), aux[0, 0], p.reshape(E)


# speedup vs baseline: 1.4085x; 1.4085x over previous
import functools

import jax
import jax.numpy as jnp
import numpy as np
from jax.experimental import pallas as pl
from jax.experimental.pallas import tpu as pltpu
from jax.experimental.pallas import tpu_sc as plsc

S, D, H, E = 2048, 768, 12, 64
DH = D // H
FF = 2 * D
TOPK = 1
CAP = int(1.25 * S / E)
EPS = 1e-6
ZW = 1e-3
NEG = -1e9
HALF = DH // 2

SBLK = 256
NS = S // SBLK
NPAD = ((E * CAP + 8) // 8) * 8

HIGH = jax.lax.Precision.HIGHEST


def _dot(a, b, dims):
    return jax.lax.dot_general(a.astype(jnp.bfloat16), b.astype(jnp.bfloat16),
                               dims, preferred_element_type=jnp.float32)


def _rsqrt(x):
    y = jax.lax.rsqrt(x)
    return y * (1.5 - 0.5 * x * y * y)


def _rms(x, g):
    return x * _rsqrt(jnp.mean(x * x, axis=-1, keepdims=True) + EPS) * g


def _rope_swap(x):
    parts = []
    for h in range(H):
        parts.append(x[:, h * DH + HALF:h * DH + DH])
        parts.append(x[:, h * DH:h * DH + HALF])
    return jnp.concatenate(parts, axis=1)


def _k123_block(x_ref, g1_ref, wqkv_ref, cos_ref, sin_ref, wo_ref, g2_ref,
                wg_ref, h_ref, xt_ref, lg_ref, ks_ref, vs_ref):
    i = pl.program_id(0)
    xn = _rms(x_ref[...], g1_ref[...])
    qkv = _dot(xn, wqkv_ref[...], (((1,), (0,)), ((), ())))
    q, k, v = qkv[:, :D], qkv[:, D:2 * D], qkv[:, 2 * D:]
    c, s = cos_ref[...], sin_ref[...]
    q = q * c + _rope_swap(q) * s
    ks_ref[pl.ds(i * SBLK, SBLK), :] = k * c + _rope_swap(k) * s
    vs_ref[pl.ds(i * SBLK, SBLK), :] = v

    row_g = jax.lax.broadcasted_iota(jnp.int32, (SBLK, S), 0) + i * SBLK
    col_g = jax.lax.broadcasted_iota(jnp.int32, (SBLK, S), 1)
    causal = col_g <= row_g
    scale = 1.0 / np.sqrt(DH)
    o_parts = []
    for h in range(H):
        sl = slice(h * DH, (h + 1) * DH)
        sc = _dot(q[:, sl], ks_ref[:, sl],
                  (((1,), (1,)), ((), ()))) * scale
        sc = jnp.where(causal, sc, NEG)
        m = jnp.max(sc, axis=1, keepdims=True)
        p = jnp.exp(sc - m)
        l = jnp.sum(p, axis=1, keepdims=True)
        o_parts.append(_dot(p, vs_ref[:, sl], (((1,), (0,)), ((), ()))) / l)
    o = jnp.concatenate(o_parts, axis=1)
    hh = x_ref[...] + _dot(o, wo_ref[...], (((1,), (0,)), ((), ())))
    xt = _rms(hh, g2_ref[...])
    h_ref[...] = hh
    xt_ref[...] = xt
    lg_ref[...] = _dot(xt, wg_ref[...], (((1,), (0,)), ((), ())))


def _k4_route(lg_ref, flat_ref, mask_ref, p_ref, aux_ref):
    CH = 128
    nch = S // CH
    r_i = jax.lax.broadcasted_iota(jnp.int32, (CH, CH), 0)
    c_i = jax.lax.broadcasted_iota(jnp.int32, (CH, CH), 1)
    tril = (r_i >= c_i).astype(jnp.float32)
    iota_e = jax.lax.broadcasted_iota(jnp.int32, (CH, E), 1)
    base = jnp.zeros((1, E), jnp.float32)
    p_acc = jnp.zeros((1, E), jnp.float32)
    lse2 = jnp.zeros((1, 1), jnp.float32)
    for c in range(nch):
        lg = lg_ref[c * CH:(c + 1) * CH, :]
        m = jnp.max(lg, axis=1, keepdims=True)
        ex = jnp.exp(lg - m)
        se = jnp.sum(ex, axis=1, keepdims=True)
        p_acc = p_acc + jnp.sum(ex / se, axis=0, keepdims=True)
        lse = m + jnp.log(se)
        lse2 = lse2 + jnp.sum(lse * lse, axis=0, keepdims=True)
        eidx = jnp.min(jnp.where(lg == m, iota_e, E), axis=1, keepdims=True)
        oh = (iota_e == eidx).astype(jnp.float32)
        csum = jax.lax.dot_general(tril, oh, (((1,), (0,)), ((), ())))
        pos = jnp.sum((csum + base) * oh, axis=1, keepdims=True).astype(
            jnp.int32) - 1
        keep = pos < CAP
        flat_ref[c * CH:(c + 1) * CH, :] = jnp.where(
            keep, eidx * CAP + pos, E * CAP)
        mask_ref[c * CH:(c + 1) * CH, :] = keep.astype(jnp.float32)
        base = base + jnp.sum(oh, axis=0, keepdims=True)
    f = base * (1.0 / S)
    p = p_acc * (1.0 / S)
    p_ref[...] = p
    aux_ref[...] = (E * jnp.sum(f * p, keepdims=True).reshape(1, 1)
                    + ZW * lse2 * (1.0 / S))


NC = 2
NW = 32
TPW = S // NW


def _sc_scatter(flat_hbm, xt_hbm, ein_hbm, idx_v, rows_v, sem):
    wid = jax.lax.axis_index("s") * NC + jax.lax.axis_index("c")
    base = wid * TPW
    pltpu.sync_copy(flat_hbm.at[pl.ds(base, TPW)], idx_v)
    pltpu.sync_copy(xt_hbm.at[pl.ds(base, TPW)], rows_v)
    pltpu.async_copy(rows_v, ein_hbm.at[idx_v], sem).wait()


def _sc_gather(flat_hbm, eout_hbm, moe_hbm, idx_v, rows_v, sem):
    wid = jax.lax.axis_index("s") * NC + jax.lax.axis_index("c")
    base = wid * TPW
    pltpu.sync_copy(flat_hbm.at[pl.ds(base, TPW)], idx_v)
    pltpu.async_copy(eout_hbm.at[idx_v], rows_v, sem).wait()
    pltpu.sync_copy(rows_v, moe_hbm.at[pl.ds(base, TPW)])


def _k6_ffn(ein_ref, w1_ref, w2_ref, eout_ref):
    x = ein_ref[...]
    h1 = _dot(x, w1_ref[0], (((1,), (0,)), ((), ())))
    h1 = jax.nn.gelu(h1)
    eout_ref[...] = _dot(h1, w2_ref[0], (((1,), (0,)), ((), ())))


def _k8_combine(h_ref, moe_ref, mask_ref, out_ref):
    out_ref[...] = h_ref[...] + jnp.where(mask_ref[...] > 0.0,
                                          moe_ref[...], 0.0)


@jax.jit
def kernel(x, ln1_g, ln2_g, Wqkv, Wo, Wg, W1, W2):
    xt2 = x.reshape(S, D)
    g1 = ln1_g.reshape(1, D)
    g2 = ln2_g.reshape(1, D)

    freqs = 1.0 / (10000.0 ** (jnp.arange(HALF, dtype=jnp.float32) / HALF))
    ang = jnp.arange(S, dtype=jnp.float32)[:, None] * freqs[None, :]
    cos_t = jnp.tile(jnp.concatenate([jnp.cos(ang)] * 2, axis=1), (1, H))
    sin_t = jnp.tile(
        jnp.concatenate([-jnp.sin(ang), jnp.sin(ang)], axis=1), (1, H))

    blk = lambda idx: pl.BlockSpec((SBLK, D), idx)
    h, xt, lg = pl.pallas_call(
        _k123_block,
        grid=(NS,),
        in_specs=[
            blk(lambda i: (i, 0)),
            pl.BlockSpec((1, D), lambda i: (0, 0)),
            pl.BlockSpec((D, 3 * D), lambda i: (0, 0)),
            blk(lambda i: (i, 0)),
            blk(lambda i: (i, 0)),
            pl.BlockSpec((D, D), lambda i: (0, 0)),
            pl.BlockSpec((1, D), lambda i: (0, 0)),
            pl.BlockSpec((D, E), lambda i: (0, 0)),
        ],
        out_specs=[blk(lambda i: (i, 0)), blk(lambda i: (i, 0)),
                   pl.BlockSpec((SBLK, E), lambda i: (i, 0))],
        out_shape=[jax.ShapeDtypeStruct((S, D), jnp.float32),
                   jax.ShapeDtypeStruct((S, D), jnp.float32),
                   jax.ShapeDtypeStruct((S, E), jnp.float32)],
        scratch_shapes=[pltpu.VMEM((S, D), jnp.float32),
                        pltpu.VMEM((S, D), jnp.float32)],
    )(xt2, g1, Wqkv, cos_t, sin_t, Wo, g2, Wg)

    flat, maskf, p, aux = pl.pallas_call(
        _k4_route,
        in_specs=[pl.BlockSpec((S, E), lambda: (0, 0))],
        out_specs=[pl.BlockSpec((S, 1), lambda: (0, 0)),
                   pl.BlockSpec((S, 1), lambda: (0, 0)),
                   pl.BlockSpec((1, E), lambda: (0, 0)),
                   pl.BlockSpec((1, 1), lambda: (0, 0))],
        out_shape=[jax.ShapeDtypeStruct((S, 1), jnp.int32),
                   jax.ShapeDtypeStruct((S, 1), jnp.float32),
                   jax.ShapeDtypeStruct((1, E), jnp.float32),
                   jax.ShapeDtypeStruct((1, 1), jnp.float32)],
    )(lg)

    flat1 = flat.reshape(S)
    sc_mesh = plsc.VectorSubcoreMesh(core_axis_name="c", subcore_axis_name="s")
    sc_scratch = [pltpu.VMEM((TPW,), jnp.int32),
                  pltpu.VMEM((TPW, D), jnp.float32),
                  pltpu.SemaphoreType.DMA]

    ein = pl.kernel(
        _sc_scatter,
        out_type=jax.ShapeDtypeStruct((NPAD, D), jnp.float32),
        mesh=sc_mesh,
        scratch_types=sc_scratch,
    )(flat1, xt)

    eout = pl.pallas_call(
        _k6_ffn,
        grid=(E,),
        in_specs=[
            pl.BlockSpec((CAP, D), lambda e: (e, 0)),
            pl.BlockSpec((1, D, FF), lambda e: (e, 0, 0)),
            pl.BlockSpec((1, FF, D), lambda e: (e, 0, 0)),
        ],
        out_specs=pl.BlockSpec((CAP, D), lambda e: (e, 0)),
        out_shape=jax.ShapeDtypeStruct((NPAD, D), jnp.float32),
    )(ein, W1, W2)

    moe = pl.kernel(
        _sc_gather,
        out_type=jax.ShapeDtypeStruct((S, D), jnp.float32),
        mesh=sc_mesh,
        scratch_types=sc_scratch,
    )(flat1, eout)

    out = pl.pallas_call(
        _k8_combine,
        grid=(NS,),
        in_specs=[blk(lambda i: (i, 0)), blk(lambda i: (i, 0)),
                  pl.BlockSpec((SBLK, 1), lambda i: (i, 0))],
        out_specs=blk(lambda i: (i, 0)),
        out_shape=jax.ShapeDtypeStruct((S, D), jnp.float32),
    )(h, moe, maskf)

    return out.reshape(1, S, D), aux[0, 0], p.reshape(E)
